# baseline (speedup 1.0000x reference)
import functools

import jax
import jax.numpy as jnp
from jax import lax
from jax.experimental import pallas as pl
from jax.experimental.pallas import tpu as pltpu
from jax.experimental.pallas import tpu_sc as plsc

CAT = 100000
LANES = 128
ROWS = 784
PAD = ROWS * LANES
B = 16384
F = 26
N = B * F
EMB = 32
NW = 32
CHUNK = N // NW
SCL = 16
B4 = N * EMB // LANES
R4 = 2048


def _distances_kernel(s_ref, o_ref):
    s = s_ref[...]
    m = jnp.max(s)
    e = jnp.exp(s - m)
    total = jnp.sum(e)
    lane = lax.broadcasted_iota(jnp.int32, (ROWS, LANES), 1)
    x = e
    for k in (1, 2, 4, 8, 16, 32, 64):
        x = x + jnp.where(lane >= k, pltpu.roll(x, k, axis=1), 0.0)
    r = x[:, LANES - 1:LANES]
    i0 = lax.broadcasted_iota(jnp.int32, (ROWS, ROWS), 0)
    i1 = lax.broadcasted_iota(jnp.int32, (ROWS, ROWS), 1)
    lmat = (i1 < i0).astype(jnp.float32)
    offs = lax.dot_general(lmat, r, (((1,), (0,)), ((), ())),
                           preferred_element_type=jnp.float32)
    table = (offs + x - e) / total
    blo = lax.bitcast_convert_type(table[:ROWS // 2], jnp.int32)
    bhi = lax.bitcast_convert_type(table[ROWS // 2:], jnp.int32)
    blo = ((blo + 0x8000) >> 16) & 0xFFFF
    bhi = (bhi + 0x8000) & ~0xFFFF
    o_ref[...] = bhi | blo


HALF = PAD // 2
NSTG = 8
STG = HALF // NSTG


def _make_sc_gather_body(idx_base, chunk):
    def body(table_hbm, idx_hbm, out_hbm, table_v, idx_v, d_v, sem, isem):
        wid = lax.axis_index("s") * 2 + lax.axis_index("c")
        base = idx_base + wid * chunk
        icp = pltpu.async_copy(idx_hbm.at[pl.ds(base, chunk)], idx_v, isem)
        cps = []
        for k in range(NSTG):
            r = (wid + k) % NSTG
            cps.append(pltpu.async_copy(table_hbm.at[pl.ds(r * STG, STG)],
                                        table_v.at[pl.ds(r * STG, STG)], sem))
        for cp in cps:
            cp.wait()
        icp.wait()

        @pl.loop(0, chunk, step=8 * SCL)
        def _(i):
            for j in range(8):
                off = i + j * SCL
                iv = idx_v[pl.ds(off, SCL)]
                hi = iv >= HALF
                jv = jnp.where(hi, iv - HALF, iv)
                word = plsc.load_gather(table_v, [jv])
                bits = jnp.where(hi, word & ~0xFFFF, word << 16)
                d_v[pl.ds(off, SCL)] = lax.bitcast_convert_type(bits, jnp.float32)

        pltpu.sync_copy(d_v, out_hbm.at[pl.ds(wid * chunk, chunk)])

    return body


@functools.cache
def _sc_gather(idx_base, count):
    chunk = count // NW
    mesh = plsc.VectorSubcoreMesh(core_axis_name="c", subcore_axis_name="s")
    cp = pltpu.CompilerParams()
    if "needs_layout_passes" in pltpu.CompilerParams.__dataclass_fields__:
        import dataclasses
        cp = dataclasses.replace(cp, needs_layout_passes=False)
    return pl.kernel(
        _make_sc_gather_body(idx_base, chunk),
        out_type=jax.ShapeDtypeStruct((count,), jnp.float32),
        mesh=mesh,
        compiler_params=cp,
        scratch_types=[
            pltpu.VMEM((HALF,), jnp.int32),
            pltpu.VMEM((chunk,), jnp.int32),
            pltpu.VMEM((chunk,), jnp.float32),
            pltpu.SemaphoreType.DMA,
            pltpu.SemaphoreType.DMA,
        ],
    )


BLK = B
F1 = 14
F2 = F - F1


def _expand_kernel(w_ref, b_ref, d_ref, o_ref):
    dd = d_ref[...]
    o_ref[...] = w_ref[...] * dd + b_ref[...]


def _expand_kernel_b(w_ref, b_ref, d_ref, prev_ref, o_ref):
    del prev_ref
    dd = d_ref[...]
    o_ref[...] = w_ref[...] * dd + b_ref[...]


def kernel(x, distance_scores, weight, bias):
    xt_flat = jnp.transpose(x).reshape(-1).astype(jnp.int32)
    s_pad = jnp.pad(distance_scores.astype(jnp.float32),
                    (0, PAD - (CAT - 1)), constant_values=-1e30)
    table2d = pl.pallas_call(
        _distances_kernel,
        out_shape=jax.ShapeDtypeStruct((ROWS // 2, LANES), jnp.int32),
    )(s_pad.reshape(ROWS, LANES))
    table = table2d.reshape(HALF)
    dt1 = _sc_gather(0, F1 * B)(table, xt_flat)
    dt2 = _sc_gather(F1 * B, F2 * B)(table, xt_flat)
    w3 = weight.astype(jnp.float32).reshape(1, EMB, 1)
    b3 = bias.astype(jnp.float32).reshape(1, EMB, 1)
    out_a = pl.pallas_call(
        _expand_kernel,
        grid=(F1 // 2,),
        in_specs=[
            pl.BlockSpec((1, EMB, 1), lambda f: (0, 0, 0)),
            pl.BlockSpec((1, EMB, 1), lambda f: (0, 0, 0)),
            pl.BlockSpec((2, 1, BLK), lambda f: (f, 0, 0)),
        ],
        out_specs=pl.BlockSpec((2, EMB, BLK), lambda f: (f, 0, 0)),
        out_shape=jax.ShapeDtypeStruct((F, EMB, B), jnp.float32),
    )(w3, b3, dt1.reshape(F1, 1, B))
    out3 = pl.pallas_call(
        _expand_kernel_b,
        grid=(F2 // 2,),
        in_specs=[
            pl.BlockSpec((1, EMB, 1), lambda f: (0, 0, 0)),
            pl.BlockSpec((1, EMB, 1), lambda f: (0, 0, 0)),
            pl.BlockSpec((2, 1, BLK), lambda f: (f, 0, 0)),
            pl.BlockSpec(memory_space=pl.ANY),
        ],
        out_specs=pl.BlockSpec((2, EMB, BLK), lambda f: (f + F1 // 2, 0, 0)),
        out_shape=jax.ShapeDtypeStruct((F, EMB, B), jnp.float32),
        input_output_aliases={3: 0},
    )(w3, b3, dt2.reshape(F2, 1, B), out_a)
    return jnp.transpose(out3, (2, 0, 1))


# speedup vs baseline: 1.1108x; 1.1108x over previous
import functools

import jax
import jax.numpy as jnp
from jax import lax
from jax.experimental import pallas as pl
from jax.experimental.pallas import tpu as pltpu
from jax.experimental.pallas import tpu_sc as plsc

CAT = 100000
LANES = 128
ROWS = 784
PAD = ROWS * LANES
B = 16384
F = 26
N = B * F
EMB = 32
NW = 32
CHUNK = N // NW
SCL = 16
B4 = N * EMB // LANES
R4 = 2048


def _distances_kernel(s_ref, o_ref):
    s = s_ref[...]
    m = jnp.max(s)
    e = jnp.exp(s - m)
    total = jnp.sum(e)
    lane = lax.broadcasted_iota(jnp.int32, (ROWS, LANES), 1)
    x = e
    for k in (1, 2, 4, 8, 16, 32, 64):
        x = x + jnp.where(lane >= k, pltpu.roll(x, k, axis=1), 0.0)
    r = x[:, LANES - 1:LANES]
    i0 = lax.broadcasted_iota(jnp.int32, (ROWS, ROWS), 0)
    i1 = lax.broadcasted_iota(jnp.int32, (ROWS, ROWS), 1)
    lmat = (i1 < i0).astype(jnp.float32)
    offs = lax.dot_general(lmat, r, (((1,), (0,)), ((), ())),
                           preferred_element_type=jnp.float32)
    table = (offs + x - e) / total
    blo = lax.bitcast_convert_type(table[:ROWS // 2], jnp.int32)
    bhi = lax.bitcast_convert_type(table[ROWS // 2:], jnp.int32)
    blo = ((blo + 0x8000) >> 16) & 0xFFFF
    bhi = (bhi + 0x8000) & ~0xFFFF
    o_ref[...] = bhi | blo


HALF = PAD // 2
NSTG = 8
STG = HALF // NSTG


def _make_sc_gather_body(idx_base, chunk):
    def body(table_hbm, idx_hbm, out_hbm, table_v, idx_v, d_v, sem, isem):
        wid = lax.axis_index("s") * 2 + lax.axis_index("c")
        base = idx_base + wid * chunk
        icp = pltpu.async_copy(idx_hbm.at[pl.ds(base, chunk)], idx_v, isem)
        cps = []
        for k in range(NSTG):
            r = (wid + k) % NSTG
            cps.append(pltpu.async_copy(table_hbm.at[pl.ds(r * STG, STG)],
                                        table_v.at[pl.ds(r * STG, STG)], sem))
        for cp in cps:
            cp.wait()
        icp.wait()

        @pl.loop(0, chunk, step=8 * SCL)
        def _(i):
            for j in range(8):
                off = i + j * SCL
                iv = idx_v[pl.ds(off, SCL)]
                hi = iv >= HALF
                jv = jnp.where(hi, iv - HALF, iv)
                word = plsc.load_gather(table_v, [jv])
                bits = jnp.where(hi, word & ~0xFFFF, word << 16)
                d_v[pl.ds(off, SCL)] = lax.bitcast_convert_type(bits, jnp.float32)

        pltpu.sync_copy(d_v, out_hbm.at[pl.ds(wid * chunk, chunk)])

    return body


@functools.cache
def _sc_gather(idx_base, count):
    chunk = count // NW
    mesh = plsc.VectorSubcoreMesh(core_axis_name="c", subcore_axis_name="s")
    cp = pltpu.CompilerParams()
    if "needs_layout_passes" in pltpu.CompilerParams.__dataclass_fields__:
        import dataclasses
        cp = dataclasses.replace(cp, needs_layout_passes=False)
    return pl.kernel(
        _make_sc_gather_body(idx_base, chunk),
        out_type=jax.ShapeDtypeStruct((count,), jnp.float32),
        mesh=mesh,
        compiler_params=cp,
        scratch_types=[
            pltpu.VMEM((HALF,), jnp.int32),
            pltpu.VMEM((chunk,), jnp.int32),
            pltpu.VMEM((chunk,), jnp.float32),
            pltpu.SemaphoreType.DMA,
            pltpu.SemaphoreType.DMA,
        ],
    )


BLK = B


def _expand_kernel(w_ref, b_ref, d_ref, o_ref):
    dd = d_ref[...]
    o_ref[...] = w_ref[...] * dd + b_ref[...]


def kernel(x, distance_scores, weight, bias):
    xt_flat = jnp.transpose(x).reshape(-1).astype(jnp.int32)
    s_pad = jnp.pad(distance_scores.astype(jnp.float32),
                    (0, PAD - (CAT - 1)), constant_values=-1e30)
    table2d = pl.pallas_call(
        _distances_kernel,
        out_shape=jax.ShapeDtypeStruct((ROWS // 2, LANES), jnp.int32),
    )(s_pad.reshape(ROWS, LANES))
    table = table2d.reshape(HALF)
    dt = _sc_gather(0, N)(table, xt_flat)
    w3 = weight.astype(jnp.float32).reshape(1, EMB, 1)
    b3 = bias.astype(jnp.float32).reshape(1, EMB, 1)
    out3 = pl.pallas_call(
        _expand_kernel,
        grid=(F // 2,),
        in_specs=[
            pl.BlockSpec((1, EMB, 1), lambda f: (0, 0, 0)),
            pl.BlockSpec((1, EMB, 1), lambda f: (0, 0, 0)),
            pl.BlockSpec((2, 1, BLK), lambda f: (f, 0, 0)),
        ],
        out_specs=pl.BlockSpec((2, EMB, BLK), lambda f: (f, 0, 0)),
        out_shape=jax.ShapeDtypeStruct((F, EMB, B), jnp.float32),
    )(w3, b3, dt.reshape(F, 1, B))
    return jnp.transpose(out3, (2, 0, 1))
